# baseline (speedup 1.0000x reference)
import functools

import jax
import jax.numpy as jnp
from jax import lax
from jax.experimental import pallas as pl
from jax.experimental.pallas import tpu as pltpu
from jax.experimental.pallas import tpu_sc as plsc

VOCAB = 100000
EMBED_DIM = 128
BATCH = 4096
HIST = 50
HIST_PAD = 56

_INFO = plsc.get_sparse_core_info()
_NC = _INFO.num_cores
_NS = _INFO.num_subcores
_NW = _NC * _NS

_BP = BATCH * HIST_PAD
_PER_W = _BP // _NW
_GB = 2
_CHUNK = _GB * HIST_PAD
_NCH = _PER_W // _CHUNK
_BATCH_PER_W = BATCH // _NW


@functools.partial(
    pl.kernel,
    mesh=plsc.VectorSubcoreMesh(core_axis_name="c", subcore_axis_name="s"),
    out_type=jax.ShapeDtypeStruct((BATCH, HIST, EMBED_DIM), jnp.float32),
    scratch_types=[
        pltpu.VMEM((_PER_W,), jnp.int32),
        pltpu.VMEM((_CHUNK, EMBED_DIM), jnp.float32),
        pltpu.SemaphoreType.DMA,
        pltpu.SemaphoreType.DMA,
    ],
    compiler_params=pltpu.CompilerParams(use_tc_tiling_on_sc=True),
)
def _gather_kernel(table_hbm, idx_hbm, out_hbm, idx_v, rows_v, gsem, ssem):
    wid = lax.axis_index("s") * _NC + lax.axis_index("c")
    base = wid * _PER_W
    bbase = wid * _BATCH_PER_W
    pltpu.sync_copy(idx_hbm.at[pl.ds(base, _PER_W)], idx_v)

    def body(j, carry):
        pltpu.async_copy(
            table_hbm.at[idx_v.at[pl.ds(j * _CHUNK, _CHUNK)]], rows_v, gsem
        ).wait()
        src = rows_v.reshape(_GB, HIST_PAD, EMBED_DIM).at[:, :HIST, :]
        pltpu.async_copy(
            src, out_hbm.at[pl.ds(bbase + j * _GB, _GB)], ssem
        ).wait()
        return carry

    lax.fori_loop(0, _NCH, body, 0)


def kernel(token_ids, table):
    ids = token_ids.astype(jnp.int32)
    pad = jnp.zeros((BATCH, HIST_PAD - HIST), jnp.int32)
    idx = jnp.concatenate([ids, pad], axis=1).reshape(_BP)
    return _gather_kernel(table, idx)


# speedup vs baseline: 1.0004x; 1.0004x over previous
import functools

import jax
import jax.numpy as jnp
from jax import lax
from jax.experimental import pallas as pl
from jax.experimental.pallas import tpu as pltpu
from jax.experimental.pallas import tpu_sc as plsc

VOCAB = 100000
EMBED_DIM = 128
BATCH = 4096
HIST = 50
HIST_PAD = 56

_INFO = plsc.get_sparse_core_info()
_NC = _INFO.num_cores
_NS = _INFO.num_subcores
_NW = _NC * _NS

_BP = BATCH * HIST_PAD
_PER_W = _BP // _NW
_GB = 2
_CHUNK = _GB * HIST_PAD
_NCH = _PER_W // _CHUNK
_BATCH_PER_W = BATCH // _NW
_NBUF = 4
_LOOK = 2


@functools.partial(
    pl.kernel,
    mesh=plsc.VectorSubcoreMesh(core_axis_name="c", subcore_axis_name="s"),
    out_type=jax.ShapeDtypeStruct((BATCH, HIST, EMBED_DIM), jnp.float32),
    scratch_types=[
        pltpu.VMEM((_PER_W,), jnp.int32),
        pltpu.VMEM((_NBUF, _CHUNK, EMBED_DIM), jnp.float32),
        pltpu.SemaphoreType.DMA((_NBUF,)),
        pltpu.SemaphoreType.DMA((_NBUF,)),
    ],
    compiler_params=pltpu.CompilerParams(use_tc_tiling_on_sc=True),
)
def _gather_kernel(table_hbm, idx_hbm, out_hbm, idx_v, rows_v, gsem, ssem):
    wid = lax.axis_index("s") * _NC + lax.axis_index("c")
    base = wid * _PER_W
    bbase = wid * _BATCH_PER_W
    pltpu.sync_copy(idx_hbm.at[pl.ds(base, _PER_W)], idx_v)

    def src_view(b):
        return rows_v.at[b].reshape(_GB, HIST_PAD, EMBED_DIM).at[:, :HIST, :]

    def issue_gather(j, b):
        pltpu.async_copy(
            table_hbm.at[idx_v.at[pl.ds(j * _CHUNK, _CHUNK)]], rows_v.at[b], gsem.at[b]
        )

    def wait_gather(b):
        pltpu.make_async_copy(
            table_hbm.at[idx_v.at[pl.ds(0, _CHUNK)]], rows_v.at[b], gsem.at[b]
        ).wait()

    def issue_store(j, b):
        pltpu.async_copy(
            src_view(b), out_hbm.at[pl.ds(bbase + j * _GB, _GB)], ssem.at[b]
        )

    def wait_store(b):
        pltpu.make_async_copy(
            src_view(b), out_hbm.at[pl.ds(bbase, _GB)], ssem.at[b]
        ).wait()

    def step(j, b, do_wait_store, do_issue_gather):
        wait_gather(b)
        issue_store(j, b)
        if do_issue_gather:
            bn = (b + _LOOK) % _NBUF
            if do_wait_store:
                wait_store(bn)
            issue_gather(j + _LOOK, bn)

    for jp in range(_LOOK):
        issue_gather(jp, jp)

    for b in range(_NBUF):
        step(b, b, b >= _NBUF - _LOOK, True)

    def body(g, carry):
        j0 = g * _NBUF
        for b in range(_NBUF):
            step(j0 + b, b, True, True)
        return carry

    lax.fori_loop(1, _NCH // _NBUF - 1, body, 0)

    for b in range(_NBUF):
        j = _NCH - _NBUF + b
        step(j, b, True, j + _LOOK < _NCH)

    for j in range(_NCH - _NBUF, _NCH):
        wait_store(j % _NBUF)


def kernel(token_ids, table):
    ids = token_ids.astype(jnp.int32)
    pad = jnp.zeros((BATCH, HIST_PAD - HIST), jnp.int32)
    idx = jnp.concatenate([ids, pad], axis=1).reshape(_BP)
    return _gather_kernel(table, idx)


# speedup vs baseline: 4.2574x; 4.2557x over previous
import functools

import jax
import jax.numpy as jnp
from jax import lax
from jax.experimental import pallas as pl
from jax.experimental.pallas import tpu as pltpu
from jax.experimental.pallas import tpu_sc as plsc

VOCAB = 100000
EMBED_DIM = 128
BATCH = 4096
HIST = 50

_INFO = plsc.get_sparse_core_info()
_NC = _INFO.num_cores
_NS = _INFO.num_subcores
_NW = _NC * _NS

_B = BATCH * HIST
_PER_W = _B // _NW
_CHUNK = 128
_NCH = _PER_W // _CHUNK
_NBUF = 5
_LOOK = 2


@functools.partial(
    pl.kernel,
    mesh=plsc.VectorSubcoreMesh(core_axis_name="c", subcore_axis_name="s"),
    out_type=jax.ShapeDtypeStruct((_B, EMBED_DIM), jnp.float32),
    scratch_types=[
        pltpu.VMEM((_PER_W,), jnp.int32),
        pltpu.VMEM((_NBUF, _CHUNK, EMBED_DIM), jnp.float32),
        pltpu.SemaphoreType.DMA((_NBUF,)),
        pltpu.SemaphoreType.DMA((_NBUF,)),
    ],
    compiler_params=pltpu.CompilerParams(use_tc_tiling_on_sc=True),
)
def _gather_kernel(table_hbm, idx_hbm, out_hbm, idx_v, rows_v, gsem, ssem):
    wid = lax.axis_index("s") * _NC + lax.axis_index("c")
    base = wid * _PER_W
    pltpu.sync_copy(idx_hbm.at[pl.ds(base, _PER_W)], idx_v)

    def issue_gather(j, b):
        pltpu.async_copy(
            table_hbm.at[idx_v.at[pl.ds(j * _CHUNK, _CHUNK)]], rows_v.at[b], gsem.at[b]
        )

    def wait_gather(b):
        pltpu.make_async_copy(
            table_hbm.at[idx_v.at[pl.ds(0, _CHUNK)]], rows_v.at[b], gsem.at[b]
        ).wait()

    def issue_store(j, b):
        pltpu.async_copy(
            rows_v.at[b], out_hbm.at[pl.ds(base + j * _CHUNK, _CHUNK)], ssem.at[b]
        )

    def wait_store(b):
        pltpu.make_async_copy(
            rows_v.at[b], out_hbm.at[pl.ds(base, _CHUNK)], ssem.at[b]
        ).wait()

    def step(j, b, do_wait_store, do_issue_gather):
        wait_gather(b)
        issue_store(j, b)
        if do_issue_gather:
            bn = (b + _LOOK) % _NBUF
            if do_wait_store:
                wait_store(bn)
            issue_gather(j + _LOOK, bn)

    for jp in range(_LOOK):
        issue_gather(jp, jp)

    for b in range(_NBUF):
        step(b, b, b >= _NBUF - _LOOK, True)

    def body(g, carry):
        j0 = g * _NBUF
        for b in range(_NBUF):
            step(j0 + b, b, True, True)
        return carry

    lax.fori_loop(1, _NCH // _NBUF - 1, body, 0)

    for b in range(_NBUF):
        j = _NCH - _NBUF + b
        step(j, b, True, j + _LOOK < _NCH)

    for j in range(_NCH - _NBUF, _NCH):
        wait_store(j % _NBUF)


def kernel(token_ids, table):
    idx = token_ids.reshape(_B).astype(jnp.int32)
    out = _gather_kernel(table, idx)
    return out.reshape(BATCH, HIST, EMBED_DIM)


# speedup vs baseline: 7.3567x; 1.7280x over previous
import functools

import jax
import jax.numpy as jnp
from jax import lax
from jax.experimental import pallas as pl
from jax.experimental.pallas import tpu as pltpu
from jax.experimental.pallas import tpu_sc as plsc

VOCAB = 100000
EMBED_DIM = 128
BATCH = 4096
HIST = 50
HIST_PAD = 56

_INFO = plsc.get_sparse_core_info()
_NC = _INFO.num_cores
_NS = _INFO.num_subcores
_NW = _NC * _NS

_BP = BATCH * HIST_PAD
_PER_W = _BP // _NW
_GB = 2
_CHUNK = _GB * HIST_PAD
_NCH = _PER_W // _CHUNK
_BATCH_PER_W = BATCH // _NW
_NBUF = 4
_LOOK = 2


@functools.partial(
    pl.kernel,
    mesh=plsc.VectorSubcoreMesh(core_axis_name="c", subcore_axis_name="s"),
    out_type=jax.ShapeDtypeStruct((BATCH, HIST, EMBED_DIM), jnp.float32),
    scratch_types=[
        pltpu.VMEM((_PER_W,), jnp.int32),
        pltpu.VMEM((_NBUF, _CHUNK, EMBED_DIM), jnp.float32),
        pltpu.SemaphoreType.DMA((_NBUF,)),
        pltpu.SemaphoreType.DMA((_NBUF,)),
    ],
    compiler_params=pltpu.CompilerParams(use_tc_tiling_on_sc=True),
)
def _gather_kernel(table_hbm, idx_hbm, out_hbm, idx_v, rows_v, gsem, ssem):
    wid = lax.axis_index("s") * _NC + lax.axis_index("c")
    base = wid * _PER_W
    bbase = wid * _BATCH_PER_W
    pltpu.sync_copy(idx_hbm.at[pl.ds(base, _PER_W)], idx_v)

    def src_view(b):
        return rows_v.at[b].reshape(_GB, HIST_PAD, EMBED_DIM).at[:, :HIST, :]

    def issue_gather(j, b):
        pltpu.async_copy(
            table_hbm.at[idx_v.at[pl.ds(j * _CHUNK, _CHUNK)]], rows_v.at[b], gsem.at[b]
        )

    def wait_gather(b):
        pltpu.make_async_copy(
            table_hbm.at[idx_v.at[pl.ds(0, _CHUNK)]], rows_v.at[b], gsem.at[b]
        ).wait()

    def issue_store(j, b):
        pltpu.async_copy(
            src_view(b), out_hbm.at[pl.ds(bbase + j * _GB, _GB)], ssem.at[b]
        )

    def wait_store(b):
        pltpu.make_async_copy(
            src_view(b), out_hbm.at[pl.ds(bbase, _GB)], ssem.at[b]
        ).wait()

    def step(j, b, do_wait_store, do_issue_gather):
        wait_gather(b)
        issue_store(j, b)
        if do_issue_gather:
            bn = (b + _LOOK) % _NBUF
            if do_wait_store:
                wait_store(bn)
            issue_gather(j + _LOOK, bn)

    for jp in range(_LOOK):
        issue_gather(jp, jp)

    for b in range(_NBUF):
        step(b, b, b >= _NBUF - _LOOK, True)

    def body(g, carry):
        j0 = g * _NBUF
        for b in range(_NBUF):
            step(j0 + b, b, True, True)
        return carry

    lax.fori_loop(1, _NCH // _NBUF - 1, body, 0)

    for b in range(_NBUF):
        j = _NCH - _NBUF + b
        step(j, b, True, j + _LOOK < _NCH)

    for j in range(_NCH - _NBUF, _NCH):
        wait_store(j % _NBUF)


def kernel(token_ids, table):
    ids = token_ids.astype(jnp.int32)
    pad = (
        jnp.arange(BATCH * (HIST_PAD - HIST), dtype=jnp.int32).reshape(
            BATCH, HIST_PAD - HIST
        )
        % VOCAB
    )
    idx = jnp.concatenate([ids, pad], axis=1).reshape(_BP)
    return _gather_kernel(table, idx)


# speedup vs baseline: 7.5485x; 1.0261x over previous
import functools

import jax
import jax.numpy as jnp
from jax import lax
from jax.experimental import pallas as pl
from jax.experimental.pallas import tpu as pltpu
from jax.experimental.pallas import tpu_sc as plsc

VOCAB = 100000
EMBED_DIM = 128
BATCH = 4096
HIST = 50

_INFO = plsc.get_sparse_core_info()
_NC = _INFO.num_cores
_NS = _INFO.num_subcores
_NW = _NC * _NS

_B = BATCH * HIST
_PER_W = _B // _NW
_GB = 4
_CROWS = _GB * HIST
_G0 = 128
_G1 = _CROWS - _G0
_NCH = _PER_W // _CROWS
_BATCH_PER_W = BATCH // _NW
_NBUF = 4
_LOOK = 2


@functools.partial(
    pl.kernel,
    mesh=plsc.VectorSubcoreMesh(core_axis_name="c", subcore_axis_name="s"),
    out_type=jax.ShapeDtypeStruct((BATCH, HIST, EMBED_DIM), jnp.float32),
    scratch_types=[
        pltpu.VMEM((_PER_W,), jnp.int32),
        pltpu.VMEM((_NBUF, _CROWS, EMBED_DIM), jnp.float32),
        pltpu.SemaphoreType.DMA((_NBUF,)),
        pltpu.SemaphoreType.DMA((_NBUF,)),
    ],
    compiler_params=pltpu.CompilerParams(use_tc_tiling_on_sc=True),
)
def _gather_kernel(table_hbm, idx_hbm, out_hbm, idx_v, rows_v, gsem, ssem):
    wid = lax.axis_index("s") * _NC + lax.axis_index("c")
    base = wid * _PER_W
    bbase = wid * _BATCH_PER_W
    pltpu.sync_copy(idx_hbm.at[pl.ds(base, _PER_W)], idx_v)

    def issue_gather(j, b):
        pltpu.async_copy(
            table_hbm.at[idx_v.at[pl.ds(j * _CROWS, _G0)]],
            rows_v.at[b].at[pl.ds(0, _G0)],
            gsem.at[b],
        )
        pltpu.async_copy(
            table_hbm.at[idx_v.at[pl.ds(j * _CROWS + _G0, _G1)]],
            rows_v.at[b].at[pl.ds(_G0, _G1)],
            gsem.at[b],
        )

    def wait_gather(b):
        pltpu.make_async_copy(
            table_hbm.at[idx_v.at[pl.ds(0, _G0)]],
            rows_v.at[b].at[pl.ds(0, _G0)],
            gsem.at[b],
        ).wait()
        pltpu.make_async_copy(
            table_hbm.at[idx_v.at[pl.ds(0, _G1)]],
            rows_v.at[b].at[pl.ds(_G0, _G1)],
            gsem.at[b],
        ).wait()

    def issue_store(j, b):
        pltpu.async_copy(
            rows_v.at[b].reshape(_GB, HIST, EMBED_DIM),
            out_hbm.at[pl.ds(bbase + j * _GB, _GB)],
            ssem.at[b],
        )

    def wait_store(b):
        pltpu.make_async_copy(
            rows_v.at[b].reshape(_GB, HIST, EMBED_DIM),
            out_hbm.at[pl.ds(bbase, _GB)],
            ssem.at[b],
        ).wait()

    def step(j, b, do_wait_store, do_issue_gather):
        wait_gather(b)
        issue_store(j, b)
        if do_issue_gather:
            bn = (b + _LOOK) % _NBUF
            if do_wait_store:
                wait_store(bn)
            issue_gather(j + _LOOK, bn)

    for jp in range(_LOOK):
        issue_gather(jp, jp)

    for b in range(_NBUF):
        step(b, b, b >= _NBUF - _LOOK, True)

    def body(g, carry):
        j0 = g * _NBUF
        for b in range(_NBUF):
            step(j0 + b, b, True, True)
        return carry

    lax.fori_loop(1, _NCH // _NBUF - 1, body, 0)

    for b in range(_NBUF):
        j = _NCH - _NBUF + b
        step(j, b, True, j + _LOOK < _NCH)

    for j in range(_NCH - _NBUF, _NCH):
        wait_store(j % _NBUF)


def kernel(token_ids, table):
    idx = token_ids.reshape(_B).astype(jnp.int32)
    return _gather_kernel(table, idx)


# speedup vs baseline: 10.2287x; 1.3551x over previous
import functools

import jax
import jax.numpy as jnp
from jax import lax
from jax.experimental import pallas as pl
from jax.experimental.pallas import tpu as pltpu
from jax.experimental.pallas import tpu_sc as plsc

VOCAB = 100000
EMBED_DIM = 128
BATCH = 4096
HIST = 50

_INFO = plsc.get_sparse_core_info()
_NC = _INFO.num_cores
_NS = _INFO.num_subcores
_NW = _NC * _NS

_B = BATCH * HIST
_PER_W = _B // _NW
_GB = 4
_CROWS = _GB * HIST
_G0 = 128
_G1 = _CROWS - _G0
_NCH = _PER_W // _CROWS
_BATCH_PER_W = BATCH // _NW
_NBUF = 4
_LOOK = 2


@functools.partial(
    pl.kernel,
    mesh=plsc.VectorSubcoreMesh(core_axis_name="c", subcore_axis_name="s"),
    out_type=jax.ShapeDtypeStruct((HIST, BATCH, EMBED_DIM), jnp.float32),
    scratch_types=[
        pltpu.VMEM((_PER_W,), jnp.int32),
        pltpu.VMEM((_NBUF, HIST, _GB, EMBED_DIM), jnp.float32),
        pltpu.SemaphoreType.DMA((_NBUF,)),
        pltpu.SemaphoreType.DMA((_NBUF,)),
    ],
)
def _gather_kernel(table_hbm, idx_hbm, out_hbm, idx_v, rows_v, gsem, ssem):
    wid = lax.axis_index("s") * _NC + lax.axis_index("c")
    base = wid * _PER_W
    bbase = wid * _BATCH_PER_W
    pltpu.sync_copy(idx_hbm.at[pl.ds(base, _PER_W)], idx_v)

    def gather_dst(b):
        return rows_v.at[b].reshape(_CROWS, EMBED_DIM)

    def issue_gather(j, b):
        pltpu.async_copy(
            table_hbm.at[idx_v.at[pl.ds(j * _CROWS, _G0)]],
            gather_dst(b).at[pl.ds(0, _G0)],
            gsem.at[b],
        )
        pltpu.async_copy(
            table_hbm.at[idx_v.at[pl.ds(j * _CROWS + _G0, _G1)]],
            gather_dst(b).at[pl.ds(_G0, _G1)],
            gsem.at[b],
        )

    def wait_gather(b):
        pltpu.make_async_copy(
            table_hbm.at[idx_v.at[pl.ds(0, _G0)]],
            gather_dst(b).at[pl.ds(0, _G0)],
            gsem.at[b],
        ).wait()
        pltpu.make_async_copy(
            table_hbm.at[idx_v.at[pl.ds(0, _G1)]],
            gather_dst(b).at[pl.ds(_G0, _G1)],
            gsem.at[b],
        ).wait()

    def issue_store(j, b):
        pltpu.async_copy(
            rows_v.at[b],
            out_hbm.at[:, pl.ds(bbase + j * _GB, _GB), :],
            ssem.at[b],
        )

    def wait_store(b):
        pltpu.make_async_copy(
            rows_v.at[b],
            out_hbm.at[:, pl.ds(bbase, _GB), :],
            ssem.at[b],
        ).wait()

    def step(j, b, do_wait_store, do_issue_gather):
        wait_gather(b)
        issue_store(j, b)
        if do_issue_gather:
            bn = (b + _LOOK) % _NBUF
            if do_wait_store:
                wait_store(bn)
            issue_gather(j + _LOOK, bn)

    for jp in range(_LOOK):
        issue_gather(jp, jp)

    for b in range(_NBUF):
        step(b, b, b >= _NBUF - _LOOK, True)

    def body(g, carry):
        j0 = g * _NBUF
        for b in range(_NBUF):
            step(j0 + b, b, True, True)
        return carry

    lax.fori_loop(1, _NCH // _NBUF - 1, body, 0)

    for b in range(_NBUF):
        j = _NCH - _NBUF + b
        step(j, b, True, j + _LOOK < _NCH)

    for j in range(_NCH - _NBUF, _NCH):
        wait_store(j % _NBUF)


def kernel(token_ids, table):
    idx = (
        token_ids.astype(jnp.int32)
        .reshape(_NW, _NCH, _GB, HIST)
        .transpose(0, 1, 3, 2)
        .reshape(_B)
    )
    out = _gather_kernel(table, idx)
    return out.transpose(1, 0, 2)


# speedup vs baseline: 13.4137x; 1.3114x over previous
import functools

import jax
import jax.numpy as jnp
from jax import lax
from jax.experimental import pallas as pl
from jax.experimental.pallas import tpu as pltpu
from jax.experimental.pallas import tpu_sc as plsc

VOCAB = 100000
EMBED_DIM = 128
BATCH = 4096
HIST = 50

_INFO = plsc.get_sparse_core_info()
_NC = _INFO.num_cores
_NS = _INFO.num_subcores
_NW = _NC * _NS

_B = BATCH * HIST
_PER_W = _B // _NW
_CHUNK = 128
_NCH = _PER_W // _CHUNK
_NBUF = 5
_LOOK = 2


@functools.partial(
    pl.kernel,
    mesh=plsc.VectorSubcoreMesh(core_axis_name="c", subcore_axis_name="s"),
    out_type=jax.ShapeDtypeStruct((_B, EMBED_DIM), jnp.float32),
    scratch_types=[
        pltpu.VMEM((_PER_W,), jnp.int32),
        pltpu.VMEM((_NBUF, _CHUNK, EMBED_DIM), jnp.float32),
        pltpu.SemaphoreType.DMA((_NBUF,)),
        pltpu.SemaphoreType.DMA((_NBUF,)),
    ],
)
def _gather_kernel(table_hbm, idx_hbm, out_hbm, idx_v, rows_v, gsem, ssem):
    wid = lax.axis_index("s") * _NC + lax.axis_index("c")
    base = wid * _PER_W
    pltpu.sync_copy(idx_hbm.at[pl.ds(base, _PER_W)], idx_v)

    def issue_gather(j, b):
        pltpu.async_copy(
            table_hbm.at[idx_v.at[pl.ds(j * _CHUNK, _CHUNK)]], rows_v.at[b], gsem.at[b]
        )

    def wait_gather(b):
        pltpu.make_async_copy(
            table_hbm.at[idx_v.at[pl.ds(0, _CHUNK)]], rows_v.at[b], gsem.at[b]
        ).wait()

    def issue_store(j, b):
        pltpu.async_copy(
            rows_v.at[b], out_hbm.at[pl.ds(base + j * _CHUNK, _CHUNK)], ssem.at[b]
        )

    def wait_store(b):
        pltpu.make_async_copy(
            rows_v.at[b], out_hbm.at[pl.ds(base, _CHUNK)], ssem.at[b]
        ).wait()

    def step(j, b, do_wait_store, do_issue_gather):
        wait_gather(b)
        issue_store(j, b)
        if do_issue_gather:
            bn = (b + _LOOK) % _NBUF
            if do_wait_store:
                wait_store(bn)
            issue_gather(j + _LOOK, bn)

    for jp in range(_LOOK):
        issue_gather(jp, jp)

    for b in range(_NBUF):
        step(b, b, b >= _NBUF - _LOOK, True)

    def body(g, carry):
        j0 = g * _NBUF
        for b in range(_NBUF):
            step(j0 + b, b, True, True)
        return carry

    lax.fori_loop(1, _NCH // _NBUF - 1, body, 0)

    for b in range(_NBUF):
        j = _NCH - _NBUF + b
        step(j, b, True, j + _LOOK < _NCH)

    for j in range(_NCH - _NBUF, _NCH):
        wait_store(j % _NBUF)


def kernel(token_ids, table):
    idx = token_ids.astype(jnp.int32).T.reshape(_B)
    out = _gather_kernel(table, idx)
    return out.reshape(HIST, BATCH, EMBED_DIM).transpose(1, 0, 2)
